# baseline (speedup 1.0000x reference)
import functools

import jax
import jax.numpy as jnp
from jax import lax
from jax.experimental import pallas as pl
from jax.experimental.pallas import tpu as pltpu
from jax.experimental.pallas import tpu_sc as plsc

_VOCAB = 100000
_EMBED = 64
_BATCH = 16384
_SEQ = 50

_NW = 32
_RPW = _BATCH // _NW
_TPW = _RPW * _SEQ
_CHUNK = 128
_NCH = _TPW // _CHUNK

_K = 50
_ROWS2 = _VOCAB // _K
_COLS2 = _K * _EMBED
_ROW_BLK = 200


def _scores_body(t_ref, w_ref, b_ref, o_ref):
    i = pl.program_id(0)
    part = jnp.sum(t_ref[...] * w_ref[...], axis=0, keepdims=True) * (1.0 / _SEQ)

    @pl.when(i == 0)
    def _init():
        o_ref[...] = part + b_ref[0] * (1.0 / _SEQ)

    @pl.when(i != 0)
    def _acc():
        o_ref[...] = o_ref[...] + part


def _tc_scores(table_t, w, b):
    return pl.pallas_call(
        _scores_body,
        grid=(_EMBED // 8,),
        in_specs=[
            pl.BlockSpec((8, _VOCAB), lambda i: (i, 0)),
            pl.BlockSpec((8, 1), lambda i: (i, 0)),
            pl.BlockSpec(memory_space=pltpu.SMEM),
        ],
        out_specs=pl.BlockSpec((1, _VOCAB), lambda i: (0, 0)),
        out_shape=jax.ShapeDtypeStruct((1, _VOCAB), jnp.float32),
    )(table_t, w, b)


@functools.cache
def _make_sc_pool():
    mesh = plsc.VectorSubcoreMesh(core_axis_name="c", subcore_axis_name="s")

    @functools.partial(
        pl.kernel,
        out_type=jax.ShapeDtypeStruct((_BATCH,), jnp.float32),
        mesh=mesh,
        scratch_types=[
            pltpu.VMEM((_SEQ, _RPW), jnp.int32),
            pltpu.VMEM((_TPW,), jnp.float32),
            pltpu.VMEM((_RPW,), jnp.float32),
            pltpu.SemaphoreType.DMA,
        ],
    )
    def _sc_pool(xt_hbm, s_hbm, out_hbm, idx_v, vals_v, res_v, sem):
        wid = lax.axis_index("s") * 2 + lax.axis_index("c")
        pltpu.sync_copy(xt_hbm.at[:, pl.ds(wid * _RPW, _RPW)], idx_v)
        def gather_step(j, carry):
            descs = [
                pltpu.async_copy(
                    s_hbm.at[idx_v.at[j, pl.ds(k * _CHUNK, _CHUNK)]],
                    vals_v.at[pl.ds(j * _RPW + k * _CHUNK, _CHUNK)],
                    sem)
                for k in range(_RPW // _CHUNK)
            ]
            for d in descs:
                d.wait()
            return carry

        lax.fori_loop(0, _SEQ, gather_step, 0)

        def body(c, carry):
            acc = jnp.zeros((16,), jnp.float32)
            for j in range(_SEQ):
                acc = acc + vals_v[pl.ds(j * _RPW + c * 16, 16)]
            res_v[pl.ds(c * 16, 16)] = 1.0 / (1.0 + jnp.exp(-acc))
            return carry

        lax.fori_loop(0, _RPW // 16, body, 0)
        pltpu.sync_copy(res_v, out_hbm.at[pl.ds(wid * _RPW, _RPW)])

    return _sc_pool


def kernel(x, table, W, b):
    scores = _tc_scores(table.T, W, b)
    pooled = _make_sc_pool()(x.T, scores.reshape(_VOCAB))
    return pooled.reshape(_BATCH, 1)


# speedup vs baseline: 1.1963x; 1.1963x over previous
import functools

import jax
import jax.numpy as jnp
from jax import lax
from jax.experimental import pallas as pl
from jax.experimental.pallas import tpu as pltpu
from jax.experimental.pallas import tpu_sc as plsc

_VOCAB = 100000
_EMBED = 64
_BATCH = 16384
_SEQ = 50

_NW = 32
_RPW = _BATCH // _NW
_TPW = _RPW * _SEQ
_CHUNK = 128
_NCH = _TPW // _CHUNK

_K = 50
_ROWS2 = _VOCAB // _K
_COLS2 = _K * _EMBED
_ROW_BLK = 200


def _scores_body(t_ref, w_ref, b_ref, o_ref):
    i = pl.program_id(0)
    part = jnp.sum(t_ref[...] * w_ref[...], axis=0, keepdims=True) * (1.0 / _SEQ)

    @pl.when(i == 0)
    def _init():
        o_ref[...] = part + b_ref[0] * (1.0 / _SEQ)

    @pl.when(i != 0)
    def _acc():
        o_ref[...] = o_ref[...] + part


def _tc_scores(table_t, w, b):
    return pl.pallas_call(
        _scores_body,
        grid=(_EMBED // 8,),
        in_specs=[
            pl.BlockSpec((8, _VOCAB), lambda i: (i, 0)),
            pl.BlockSpec((8, 1), lambda i: (i, 0)),
            pl.BlockSpec(memory_space=pltpu.SMEM),
        ],
        out_specs=pl.BlockSpec((1, _VOCAB), lambda i: (0, 0)),
        out_shape=jax.ShapeDtypeStruct((1, _VOCAB), jnp.float32),
    )(table_t, w, b)


@functools.cache
def _make_sc_pool():
    mesh = plsc.VectorSubcoreMesh(core_axis_name="c", subcore_axis_name="s")

    @functools.partial(
        pl.kernel,
        out_type=jax.ShapeDtypeStruct((_BATCH,), jnp.float32),
        mesh=mesh,
        scratch_types=[
            pltpu.VMEM((_SEQ, _RPW), jnp.int32),
            pltpu.VMEM((_TPW,), jnp.float32),
            pltpu.VMEM((_RPW,), jnp.float32),
            pltpu.SemaphoreType.DMA,
        ],
    )
    def _sc_pool(xt_hbm, s_hbm, out_hbm, idx_v, vals_v, res_v, sem):
        wid = lax.axis_index("s") * 2 + lax.axis_index("c")
        pltpu.sync_copy(xt_hbm.at[:, pl.ds(wid * _RPW, _RPW)], idx_v)
        def gather_step(s, carry):
            descs = []
            for k in range(8):
                t = s * 8 + k
                descs.append(pltpu.async_copy(
                    s_hbm.at[idx_v.at[t >> 2, pl.ds((t & 3) * _CHUNK, _CHUNK)]],
                    vals_v.at[pl.ds(t * _CHUNK, _CHUNK)],
                    sem))
            for d in descs:
                d.wait()
            return carry

        lax.fori_loop(0, _NCH // 8, gather_step, 0)

        def body(c, carry):
            acc = jnp.zeros((16,), jnp.float32)
            for j in range(_SEQ):
                acc = acc + vals_v[pl.ds(j * _RPW + c * 16, 16)]
            res_v[pl.ds(c * 16, 16)] = 1.0 / (1.0 + jnp.exp(-acc))
            return carry

        lax.fori_loop(0, _RPW // 16, body, 0)
        pltpu.sync_copy(res_v, out_hbm.at[pl.ds(wid * _RPW, _RPW)])

    return _sc_pool


def kernel(x, table, W, b):
    scores = _tc_scores(table.T, W, b)
    pooled = _make_sc_pool()(x.T, scores.reshape(_VOCAB))
    return pooled.reshape(_BATCH, 1)


# speedup vs baseline: 1.3150x; 1.0992x over previous
import functools

import jax
import jax.numpy as jnp
from jax import lax
from jax.experimental import pallas as pl
from jax.experimental.pallas import tpu as pltpu
from jax.experimental.pallas import tpu_sc as plsc

_VOCAB = 100000
_EMBED = 64
_BATCH = 16384
_SEQ = 50

_NW = 32
_RPW = _BATCH // _NW
_TPW = _RPW * _SEQ
_CHUNK = 128
_NCH = _TPW // _CHUNK

_K = 50
_ROWS2 = _VOCAB // _K
_COLS2 = _K * _EMBED
_ROW_BLK = 200


def _scores_body(t_ref, w_ref, b_ref, o_ref):
    i = pl.program_id(0)
    part = jnp.sum(t_ref[...] * w_ref[...], axis=0, keepdims=True) * (1.0 / _SEQ)

    @pl.when(i == 0)
    def _init():
        o_ref[...] = part + b_ref[0] * (1.0 / _SEQ)

    @pl.when(i != 0)
    def _acc():
        o_ref[...] = o_ref[...] + part


def _tc_scores(table_t, w, b):
    return pl.pallas_call(
        _scores_body,
        grid=(_EMBED // 8,),
        in_specs=[
            pl.BlockSpec((8, _VOCAB), lambda i: (i, 0)),
            pl.BlockSpec((8, 1), lambda i: (i, 0)),
            pl.BlockSpec(memory_space=pltpu.SMEM),
        ],
        out_specs=pl.BlockSpec((1, _VOCAB), lambda i: (0, 0)),
        out_shape=jax.ShapeDtypeStruct((1, _VOCAB), jnp.float32),
    )(table_t, w, b)


@functools.cache
def _make_sc_pool():
    mesh = plsc.VectorSubcoreMesh(core_axis_name="c", subcore_axis_name="s")

    @functools.partial(
        pl.kernel,
        out_type=jax.ShapeDtypeStruct((_BATCH,), jnp.float32),
        mesh=mesh,
        scratch_types=[
            pltpu.VMEM((_SEQ, _RPW), jnp.int32),
            pltpu.VMEM((_TPW,), jnp.float32),
            pltpu.VMEM((_RPW,), jnp.float32),
            pltpu.SemaphoreType.DMA,
        ],
    )
    def _sc_pool(xt_hbm, s_hbm, out_hbm, idx_v, vals_v, res_v, sem):
        wid = lax.axis_index("s") * 2 + lax.axis_index("c")
        pltpu.sync_copy(xt_hbm.at[:, pl.ds(wid * _RPW, _RPW)], idx_v)
        def _fire(s):
            for k in range(8):
                t = s * 8 + k
                pltpu.async_copy(
                    s_hbm.at[idx_v.at[t >> 2, pl.ds((t & 3) * _CHUNK, _CHUNK)]],
                    vals_v.at[pl.ds(t * _CHUNK, _CHUNK)],
                    sem)

        def _drain():
            for k in range(8):
                pltpu.make_async_copy(
                    s_hbm.at[pl.ds(0, _CHUNK)],
                    vals_v.at[pl.ds(k * _CHUNK, _CHUNK)],
                    sem).wait()

        _fire(0)

        def gather_step(s, carry):
            _fire(s)
            _drain()
            return carry

        lax.fori_loop(1, _NCH // 8, gather_step, 0)
        _drain()

        def body(c, carry):
            acc = jnp.zeros((16,), jnp.float32)
            for j in range(_SEQ):
                acc = acc + vals_v[pl.ds(j * _RPW + c * 16, 16)]
            res_v[pl.ds(c * 16, 16)] = 1.0 / (1.0 + jnp.exp(-acc))
            return carry

        lax.fori_loop(0, _RPW // 16, body, 0)
        pltpu.sync_copy(res_v, out_hbm.at[pl.ds(wid * _RPW, _RPW)])

    return _sc_pool


def kernel(x, table, W, b):
    scores = _tc_scores(table.T, W, b)
    pooled = _make_sc_pool()(x.T, scores.reshape(_VOCAB))
    return pooled.reshape(_BATCH, 1)


# speedup vs baseline: 1.3232x; 1.0062x over previous
import functools

import jax
import jax.numpy as jnp
from jax import lax
from jax.experimental import pallas as pl
from jax.experimental.pallas import tpu as pltpu
from jax.experimental.pallas import tpu_sc as plsc

_VOCAB = 100000
_EMBED = 64
_BATCH = 16384
_SEQ = 50

_NW = 32
_RPW = _BATCH // _NW
_TPW = _RPW * _SEQ
_CHUNK = 128
_NCH = _TPW // _CHUNK

_K = 50
_ROWS2 = _VOCAB // _K
_COLS2 = _K * _EMBED
_ROW_BLK = 200


def _scores_body(t_ref, w_ref, b_ref, o_ref):
    i = pl.program_id(0)
    part = jnp.sum(t_ref[...] * w_ref[...], axis=0, keepdims=True) * (1.0 / _SEQ)

    @pl.when(i == 0)
    def _init():
        o_ref[...] = part + b_ref[0] * (1.0 / _SEQ)

    @pl.when(i != 0)
    def _acc():
        o_ref[...] = o_ref[...] + part


def _tc_scores(table_t, w, b):
    return pl.pallas_call(
        _scores_body,
        grid=(_EMBED // 8,),
        in_specs=[
            pl.BlockSpec((8, _VOCAB), lambda i: (i, 0)),
            pl.BlockSpec((8, 1), lambda i: (i, 0)),
            pl.BlockSpec(memory_space=pltpu.SMEM),
        ],
        out_specs=pl.BlockSpec((1, _VOCAB), lambda i: (0, 0)),
        out_shape=jax.ShapeDtypeStruct((1, _VOCAB), jnp.float32),
    )(table_t, w, b)


@functools.cache
def _make_sc_pool():
    mesh = plsc.VectorSubcoreMesh(core_axis_name="c", subcore_axis_name="s")

    @functools.partial(
        pl.kernel,
        out_type=jax.ShapeDtypeStruct((_BATCH,), jnp.float32),
        mesh=mesh,
        scratch_types=[
            pltpu.VMEM((_SEQ, _RPW), jnp.int32),
            pltpu.VMEM((_TPW,), jnp.float32),
            pltpu.VMEM((_RPW,), jnp.float32),
            pltpu.SemaphoreType.DMA,
        ],
    )
    def _sc_pool(xt_hbm, s_hbm, out_hbm, idx_v, vals_v, res_v, sem):
        wid = lax.axis_index("s") * 2 + lax.axis_index("c")
        pltpu.sync_copy(xt_hbm.at[:, pl.ds(wid * _RPW, _RPW)], idx_v)
        def _fire(s):
            for k in range(8):
                t = s * 8 + k
                pltpu.async_copy(
                    s_hbm.at[idx_v.at[t >> 2, pl.ds((t & 3) * _CHUNK, _CHUNK)]],
                    vals_v.at[pl.ds(t * _CHUNK, _CHUNK)],
                    sem)

        def _drain():
            for k in range(8):
                pltpu.make_async_copy(
                    s_hbm.at[pl.ds(0, _CHUNK)],
                    vals_v.at[pl.ds(k * _CHUNK, _CHUNK)],
                    sem).wait()

        _fire(0)
        _fire(1)

        def gather_step(s, carry):
            _fire(s)
            _drain()
            return carry

        lax.fori_loop(2, _NCH // 8, gather_step, 0)
        _drain()
        _drain()

        def body(c, carry):
            acc = jnp.zeros((16,), jnp.float32)
            for j in range(_SEQ):
                acc = acc + vals_v[pl.ds(j * _RPW + c * 16, 16)]
            res_v[pl.ds(c * 16, 16)] = 1.0 / (1.0 + jnp.exp(-acc))
            return carry

        lax.fori_loop(0, _RPW // 16, body, 0)
        pltpu.sync_copy(res_v, out_hbm.at[pl.ds(wid * _RPW, _RPW)])

    return _sc_pool


def kernel(x, table, W, b):
    scores = _tc_scores(table.T, W, b)
    pooled = _make_sc_pool()(x.T, scores.reshape(_VOCAB))
    return pooled.reshape(_BATCH, 1)


# speedup vs baseline: 2.1037x; 1.5898x over previous
import functools

import jax
import jax.numpy as jnp
from jax import lax
from jax.experimental import pallas as pl
from jax.experimental.pallas import tpu as pltpu
from jax.experimental.pallas import tpu_sc as plsc

_VOCAB = 100000
_EMBED = 64
_BATCH = 16384
_SEQ = 50

_NW = 32
_RPW = _BATCH // _NW
_TPW = _RPW * _SEQ
_CHUNK = 128
_NCH = _TPW // _CHUNK

_K = 50
_ROWS2 = _VOCAB // _K
_COLS2 = _K * _EMBED
_ROW_BLK = 200


def _scores_body(t_ref, w_ref, b_ref, o_ref):
    i = pl.program_id(0)
    part = jnp.sum(t_ref[...] * w_ref[...], axis=0, keepdims=True) * (1.0 / _SEQ)

    @pl.when(i == 0)
    def _init():
        o_ref[...] = part + b_ref[0] * (1.0 / _SEQ)

    @pl.when(i != 0)
    def _acc():
        o_ref[...] = o_ref[...] + part


def _tc_scores(table_t, w, b):
    return pl.pallas_call(
        _scores_body,
        grid=(_EMBED // 8,),
        in_specs=[
            pl.BlockSpec((8, _VOCAB), lambda i: (i, 0)),
            pl.BlockSpec((8, 1), lambda i: (i, 0)),
            pl.BlockSpec(memory_space=pltpu.SMEM),
        ],
        out_specs=pl.BlockSpec((1, _VOCAB), lambda i: (0, 0)),
        out_shape=jax.ShapeDtypeStruct((1, _VOCAB), jnp.float32),
    )(table_t, w, b)


@functools.cache
def _make_sc_pool():
    mesh = plsc.VectorSubcoreMesh(core_axis_name="c", subcore_axis_name="s")

    @functools.partial(
        pl.kernel,
        out_type=jax.ShapeDtypeStruct((_BATCH,), jnp.float32),
        mesh=mesh,
        scratch_types=[
            pltpu.VMEM((_SEQ, _RPW), jnp.int32),
            pltpu.VMEM((_TPW,), jnp.float32),
            pltpu.VMEM((_RPW,), jnp.float32),
            pltpu.VMEM_SHARED((_VOCAB,), jnp.float32),
            pltpu.SemaphoreType.DMA,
        ],
    )
    def _sc_pool(xt_hbm, s_hbm, out_hbm, idx_v, vals_v, res_v, s_sh, sem):
        wid = lax.axis_index("s") * 2 + lax.axis_index("c")

        @pl.when(lax.axis_index("s") == 0)
        def _stage_scores():
            pltpu.sync_copy(s_hbm, s_sh)

        pltpu.sync_copy(xt_hbm.at[:, pl.ds(wid * _RPW, _RPW)], idx_v)
        plsc.subcore_barrier()
        def _fire(s):
            for k in range(8):
                t = s * 8 + k
                pltpu.async_copy(
                    s_sh.at[idx_v.at[t >> 2, pl.ds((t & 3) * _CHUNK, _CHUNK)]],
                    vals_v.at[pl.ds(t * _CHUNK, _CHUNK)],
                    sem)

        def _drain():
            for k in range(8):
                pltpu.make_async_copy(
                    s_hbm.at[pl.ds(0, _CHUNK)],
                    vals_v.at[pl.ds(k * _CHUNK, _CHUNK)],
                    sem).wait()

        _fire(0)
        _fire(1)

        def gather_step(s, carry):
            _fire(s)
            _drain()
            return carry

        lax.fori_loop(2, _NCH // 8, gather_step, 0)
        _drain()
        _drain()

        def body(c, carry):
            acc = jnp.zeros((16,), jnp.float32)
            for j in range(_SEQ):
                acc = acc + vals_v[pl.ds(j * _RPW + c * 16, 16)]
            res_v[pl.ds(c * 16, 16)] = 1.0 / (1.0 + jnp.exp(-acc))
            return carry

        lax.fori_loop(0, _RPW // 16, body, 0)
        pltpu.sync_copy(res_v, out_hbm.at[pl.ds(wid * _RPW, _RPW)])

    return _sc_pool


def kernel(x, table, W, b):
    scores = _tc_scores(table.T, W, b)
    pooled = _make_sc_pool()(x.T, scores.reshape(_VOCAB))
    return pooled.reshape(_BATCH, 1)


# speedup vs baseline: 2.3271x; 1.1062x over previous
import functools

import jax
import jax.numpy as jnp
from jax import lax
from jax.experimental import pallas as pl
from jax.experimental.pallas import tpu as pltpu
from jax.experimental.pallas import tpu_sc as plsc

_VOCAB = 100000
_EMBED = 64
_BATCH = 16384
_SEQ = 50

_NW = 32
_RPW = _BATCH // _NW
_TPW = _RPW * _SEQ
_CHUNK = 128
_NCH = _TPW // _CHUNK

_K = 50
_ROWS2 = _VOCAB // _K
_COLS2 = _K * _EMBED
_ROW_BLK = 200


def _scores_body(t_ref, w_ref, b_ref, o_ref):
    i = pl.program_id(0)
    part = lax.dot_general(
        w_ref[...], t_ref[...], (((0,), (0,)), ((), ())),
        preferred_element_type=jnp.float32) * (1.0 / _SEQ)

    @pl.when(i == 0)
    def _init():
        o_ref[...] = part + b_ref[0] * (1.0 / _SEQ)

    @pl.when(i != 0)
    def _acc():
        o_ref[...] = o_ref[...] + part


def _tc_scores(table_t, w, b):
    return pl.pallas_call(
        _scores_body,
        grid=(4,),
        in_specs=[
            pl.BlockSpec((_EMBED // 4, _VOCAB), lambda i: (i, 0)),
            pl.BlockSpec((_EMBED // 4, 1), lambda i: (i, 0)),
            pl.BlockSpec(memory_space=pltpu.SMEM),
        ],
        out_specs=pl.BlockSpec((1, _VOCAB), lambda i: (0, 0)),
        out_shape=jax.ShapeDtypeStruct((1, _VOCAB), jnp.float32),
    )(table_t, w, b)


@functools.cache
def _make_sc_pool():
    mesh = plsc.VectorSubcoreMesh(core_axis_name="c", subcore_axis_name="s")

    @functools.partial(
        pl.kernel,
        out_type=jax.ShapeDtypeStruct((_BATCH,), jnp.float32),
        mesh=mesh,
        scratch_types=[
            pltpu.VMEM((_SEQ, _RPW), jnp.int32),
            pltpu.VMEM((_TPW,), jnp.float32),
            pltpu.VMEM((_RPW,), jnp.float32),
            pltpu.VMEM_SHARED((_VOCAB,), jnp.float32),
            pltpu.SemaphoreType.DMA,
        ],
    )
    def _sc_pool(xt_hbm, s_hbm, out_hbm, idx_v, vals_v, res_v, s_sh, sem):
        wid = lax.axis_index("s") * 2 + lax.axis_index("c")

        @pl.when(lax.axis_index("s") == 0)
        def _stage_scores():
            pltpu.sync_copy(s_hbm, s_sh)

        pltpu.sync_copy(xt_hbm.at[:, pl.ds(wid * _RPW, _RPW)], idx_v)
        plsc.subcore_barrier()
        def _fire(s):
            for k in range(8):
                t = s * 8 + k
                pltpu.async_copy(
                    s_sh.at[idx_v.at[t >> 2, pl.ds((t & 3) * _CHUNK, _CHUNK)]],
                    vals_v.at[pl.ds(t * _CHUNK, _CHUNK)],
                    sem)

        def _drain():
            for k in range(8):
                pltpu.make_async_copy(
                    s_hbm.at[pl.ds(0, _CHUNK)],
                    vals_v.at[pl.ds(k * _CHUNK, _CHUNK)],
                    sem).wait()

        _fire(0)
        _fire(1)

        def gather_step(s, carry):
            _fire(s)
            _drain()
            return carry

        lax.fori_loop(2, _NCH // 8, gather_step, 0)
        _drain()
        _drain()

        def body(c, carry):
            acc = jnp.zeros((16,), jnp.float32)
            for j in range(_SEQ):
                acc = acc + vals_v[pl.ds(j * _RPW + c * 16, 16)]
            res_v[pl.ds(c * 16, 16)] = 1.0 / (1.0 + jnp.exp(-acc))
            return carry

        lax.fori_loop(0, _RPW // 16, body, 0)
        pltpu.sync_copy(res_v, out_hbm.at[pl.ds(wid * _RPW, _RPW)])

    return _sc_pool


def kernel(x, table, W, b):
    scores = _tc_scores(table.T, W, b)
    pooled = _make_sc_pool()(x.T, scores.reshape(_VOCAB))
    return pooled.reshape(_BATCH, 1)
